# Initial kernel scaffold; baseline (speedup 1.0000x reference)
import jax
import jax.numpy as jnp
from jax.experimental import pallas as pl


def kernel(pos_w, pos_v, neg_v, W_w, W_v):
    raise NotImplementedError("write your pallas kernel here")


# speedup vs baseline: 4.5513x; 4.5513x over previous
import functools

import jax
import jax.numpy as jnp
from jax import lax
from jax.experimental import pallas as pl
from jax.experimental.pallas import tpu as pltpu
from jax.experimental.pallas import tpu_sc as plsc

_LANES = 16
_NW = 32
_SUB = 16
_ICH = 80


def _make_scores_kernel(B, K, D, BPW, NSUB, NICH, NCH, CH, NBLK):
    mesh = plsc.VectorSubcoreMesh(core_axis_name="c", subcore_axis_name="s")
    nseg = D // _LANES

    @functools.partial(
        pl.kernel,
        out_type=jax.ShapeDtypeStruct((K + 1, _NW, NBLK, _LANES), jnp.float32),
        mesh=mesh,
        compiler_params=pltpu.CompilerParams(use_tc_tiling_on_sc=False),
        scratch_types=[
            pltpu.VMEM((NCH, CH), jnp.int32),
            pltpu.VMEM((NCH, CH), jnp.int32),
            pltpu.VMEM((NSUB, NICH, _ICH), jnp.int32),
            pltpu.VMEM((BPW, D), jnp.float32),
            pltpu.VMEM((BPW, D), jnp.float32),
            pltpu.VMEM((_SUB * K, D), jnp.float32),
            pltpu.VMEM((K + 1, NBLK, _LANES), jnp.float32),
            pltpu.SemaphoreType.DMA,
            pltpu.SemaphoreType.DMA,
        ],
    )
    def scores_kernel(pw_hbm, pv_hbm, nidx_hbm, ww_hbm, wv_hbm, out_hbm,
                      idxw, idxv, idxn, wrow, vrow, nrow, outv, sem1, sem2):
        wid = lax.axis_index("s") * 2 + lax.axis_index("c")

        pltpu.sync_copy(pw_hbm.at[wid], idxw)
        pltpu.sync_copy(pv_hbm.at[wid], idxv)
        pltpu.sync_copy(nidx_hbm.at[wid], idxn)

        cw = [pltpu.async_copy(ww_hbm.at[idxw.at[c]],
                               wrow.at[pl.ds(c * CH, CH)], sem1)
              for c in range(NCH)]
        cv = [pltpu.async_copy(wv_hbm.at[idxv.at[c]],
                               vrow.at[pl.ds(c * CH, CH)], sem2)
              for c in range(NCH)]
        for c in cw:
            c.wait()
        for c in cv:
            c.wait()

        lane = lax.iota(jnp.int32, _LANES)
        perms = [lane ^ m for m in (8, 4, 2, 1)]

        def lanesum(x):
            for p in perms:
                x = x + x.at[p].get(mode="promise_in_bounds")
            return x

        def dot_row(rows, rb, vb):
            acc = (rows[rb, pl.ds(0, _LANES)]
                   * vrow[vb, pl.ds(0, _LANES)])
            for j in range(1, nseg):
                acc = acc + (rows[rb, pl.ds(j * _LANES, _LANES)]
                             * vrow[vb, pl.ds(j * _LANES, _LANES)])
            return lanesum(acc)

        def pos_body(b, carry):
            s = dot_row(wrow, b, b)
            blk = b >> 4
            r = b & (_LANES - 1)
            outv[0, blk, :] = jnp.where(lane == r, s, outv[0, blk, :])
            return carry

        lax.fori_loop(0, BPW, pos_body, 0)

        def sub_body(s, carry):
            cn = [pltpu.async_copy(wv_hbm.at[idxn.at[s, c]],
                                   nrow.at[pl.ds(c * _ICH, _ICH)], sem1)
                  for c in range(NICH)]
            for c in cn:
                c.wait()

            def row_body(r, c1):
                g = s * _SUB + r

                def k_body(k, c2):
                    d = dot_row(nrow, r * K + k, g)
                    outv[k + 1, s, :] = jnp.where(lane == r, -d,
                                                  outv[k + 1, s, :])
                    return c2

                lax.fori_loop(0, K, k_body, 0)
                return c1

            lax.fori_loop(0, _SUB, row_body, 0)
            return carry

        lax.fori_loop(0, NSUB, sub_body, 0)

        pltpu.sync_copy(outv, out_hbm.at[:, wid])

    return scores_kernel


def _loss_body(s_ref, o_ref):
    x = s_ref[...]
    ls = jnp.minimum(x, 0.0) - jnp.log1p(jnp.exp(-jnp.abs(x)))
    o_ref[0, 0] = -jnp.sum(ls)


def kernel(pos_w, pos_v, neg_v, W_w, W_v):
    B, K = neg_v.shape
    D = W_w.shape[1]
    BPW = B // _NW
    CH = 128
    NCH = BPW // CH
    NBLK = BPW // _LANES
    NSUB = BPW // _SUB
    NICH = (_SUB * K) // _ICH

    pw = pos_w.reshape(_NW, NCH, CH)
    pv = pos_v.reshape(_NW, NCH, CH)
    nidx = neg_v.reshape(_NW, NSUB, NICH, _ICH)

    scores = _make_scores_kernel(B, K, D, BPW, NSUB, NICH, NCH, CH, NBLK)(
        pw, pv, nidx, W_w, W_v)

    loss = pl.pallas_call(
        _loss_body,
        out_shape=jax.ShapeDtypeStruct((1, 1), jnp.float32),
        out_specs=pl.BlockSpec(memory_space=pltpu.SMEM),
    )(scores.reshape(K + 1, B))
    return loss[0, 0]


# speedup vs baseline: 5.4183x; 1.1905x over previous
import functools

import jax
import jax.numpy as jnp
from jax import lax
from jax.experimental import pallas as pl
from jax.experimental.pallas import tpu as pltpu
from jax.experimental.pallas import tpu_sc as plsc

_LANES = 16
_NW = 32
_SUB = 16


def _make_scores_kernel(B, K, D, BPW, NSUB, CHUNKS):
    mesh = plsc.VectorSubcoreMesh(core_axis_name="c", subcore_axis_name="s")
    nseg = D // _LANES
    crows = _SUB * (K + 1)

    @functools.partial(
        pl.kernel,
        out_type=jax.ShapeDtypeStruct((K + 1, _NW, NSUB, _LANES), jnp.float32),
        mesh=mesh,
        compiler_params=pltpu.CompilerParams(use_tc_tiling_on_sc=False),
        scratch_types=[
            pltpu.VMEM((NSUB, _SUB), jnp.int32),
            pltpu.VMEM((NSUB, crows), jnp.int32),
            pltpu.VMEM((_SUB, D), jnp.float32),
            pltpu.VMEM((_SUB, D), jnp.float32),
            pltpu.VMEM((crows, D), jnp.float32),
            pltpu.VMEM((crows, D), jnp.float32),
            pltpu.VMEM((K + 1, NSUB, _LANES), jnp.float32),
            pltpu.SemaphoreType.DMA,
            pltpu.SemaphoreType.DMA,
        ],
    )
    def scores_kernel(pw_hbm, cidx_hbm, ww_hbm, wv_hbm, out_hbm,
                      idxw, idxc, wb0, wb1, cb0, cb1, outv, semA, semB):
        wid = lax.axis_index("s") * 2 + lax.axis_index("c")

        pltpu.sync_copy(pw_hbm.at[wid], idxw)
        pltpu.sync_copy(cidx_hbm.at[wid], idxc)

        def fire_set(s, wb, cb, sem):
            pltpu.async_copy(ww_hbm.at[idxw.at[s]], wb, sem)
            for off, n in CHUNKS:
                pltpu.async_copy(wv_hbm.at[idxc.at[s, pl.ds(off, n)]],
                                 cb.at[pl.ds(off, n)], sem)

        def wait_set(wb, cb, sem):
            pltpu.make_async_copy(ww_hbm.at[idxw.at[0]], wb, sem).wait()
            for off, n in CHUNKS:
                pltpu.make_async_copy(wv_hbm.at[idxc.at[0, pl.ds(off, n)]],
                                      cb.at[pl.ds(off, n)], sem).wait()

        lane = lax.iota(jnp.int32, _LANES)
        perms = [lane ^ m for m in (8, 4, 2, 1)]
        zero = jnp.zeros((_LANES,), jnp.float32)

        def lanesum(x):
            for p in perms:
                x = x + x.at[p].get(mode="promise_in_bounds")
            return x

        def compute_sub(s, wb, cb):
            def row_body(r, accs):
                vs = [cb[r, pl.ds(j * _LANES, _LANES)] for j in range(nseg)]

                def dot(rows, rb):
                    acc = rows[rb, pl.ds(0, _LANES)] * vs[0]
                    for j in range(1, nseg):
                        acc = acc + rows[rb, pl.ds(j * _LANES, _LANES)] * vs[j]
                    return lanesum(acc)

                sel = lane == r
                new = [jnp.where(sel, dot(wb, r), accs[0])]
                for k in range(K):
                    d = dot(cb, _SUB + r * K + k)
                    new.append(jnp.where(sel, -d, accs[k + 1]))
                return tuple(new)

            accs = lax.fori_loop(0, _SUB, row_body, (zero,) * (K + 1))
            for k in range(K + 1):
                outv[k, s, :] = accs[k]

        fire_set(0, wb0, cb0, semA)
        fire_set(1, wb1, cb1, semB)

        def pair_body(p, carry):
            s0 = 2 * p
            wait_set(wb0, cb0, semA)
            compute_sub(s0, wb0, cb0)

            @pl.when(s0 + 2 < NSUB)
            def _():
                fire_set(s0 + 2, wb0, cb0, semA)

            wait_set(wb1, cb1, semB)
            compute_sub(s0 + 1, wb1, cb1)

            @pl.when(s0 + 3 < NSUB)
            def _():
                fire_set(s0 + 3, wb1, cb1, semB)

            return carry

        lax.fori_loop(0, NSUB // 2, pair_body, 0)

        pltpu.sync_copy(outv, out_hbm.at[:, wid])

    return scores_kernel


def _loss_body(s_ref, o_ref):
    x = s_ref[...]
    ls = jnp.minimum(x, 0.0) - jnp.log1p(jnp.exp(-jnp.abs(x)))
    o_ref[0, 0] = -jnp.sum(ls)


def kernel(pos_w, pos_v, neg_v, W_w, W_v):
    B, K = neg_v.shape
    D = W_w.shape[1]
    BPW = B // _NW
    NSUB = BPW // _SUB
    crows = _SUB * (K + 1)
    CHUNKS = []
    off = 0
    while off < crows:
        n = min(128, crows - off)
        CHUNKS.append((off, n))
        off += n
    CHUNKS = tuple(CHUNKS)

    pw = pos_w.reshape(_NW, NSUB, _SUB)
    cidx = jnp.concatenate(
        [pos_v.reshape(_NW, NSUB, _SUB),
         neg_v.reshape(_NW, NSUB, _SUB * K)], axis=2)

    scores = _make_scores_kernel(B, K, D, BPW, NSUB, CHUNKS)(
        pw, cidx, W_w, W_v)

    loss = pl.pallas_call(
        _loss_body,
        out_shape=jax.ShapeDtypeStruct((1, 1), jnp.float32),
        out_specs=pl.BlockSpec(memory_space=pltpu.SMEM),
    )(scores.reshape(K + 1, B))
    return loss[0, 0]
